# Initial kernel scaffold; baseline (speedup 1.0000x reference)
import functools

import jax
import jax.numpy as jnp
from jax import lax
from jax.experimental import pallas as pl
from jax.experimental.pallas import tpu as pltpu
from jax.experimental.pallas import tpu_sc as plsc

NC = 2
NS = 16
L = 16
CHUNK = 128
ORDER = 8


def _prop_sc(n_pad, f, e_pad, x_pad, src, dst, w):
    fh = f // NC
    rows_per_tile = n_pad // NS
    edges_per_tile = e_pad // NS
    n_chunks = edges_per_tile // CHUNK
    mesh = plsc.VectorSubcoreMesh(
        core_axis_name="c", subcore_axis_name="s", num_cores=NC, num_subcores=NS
    )

    @functools.partial(
        pl.kernel,
        out_type=jax.ShapeDtypeStruct((n_pad, f), jnp.float32),
        mesh=mesh,
        scratch_types=[
            pltpu.VMEM_SHARED((n_pad, fh), jnp.float32),
            pltpu.VMEM_SHARED((n_pad, fh), jnp.float32),
            pltpu.VMEM((rows_per_tile, fh), jnp.float32),
            pltpu.VMEM((rows_per_tile, fh), jnp.float32),
            pltpu.VMEM((CHUNK, fh), jnp.float32),
            pltpu.VMEM((CHUNK,), jnp.int32),
            pltpu.VMEM((CHUNK,), jnp.int32),
            pltpu.VMEM((CHUNK,), jnp.float32),
        ],
    )
    def prop_kernel(x_hbm, src_hbm, dst_hbm, w_hbm, out_hbm,
                    buf_a, buf_b, prop, stage, rows, sidx, didx, wbuf):
        c = lax.axis_index("c")
        s = lax.axis_index("s")
        fbase = c * fh
        r0 = s * rows_per_tile
        row_slice = pl.ds(r0, rows_per_tile)

        pltpu.sync_copy(x_hbm.at[row_slice, pl.ds(fbase, fh)], prop)
        pltpu.sync_copy(prop, buf_a.at[row_slice])

        @pl.loop(0, rows_per_tile)
        def _(i):
            for j in range(fh // L):
                stage[i, pl.ds(j * L, L)] = jnp.zeros((L,), jnp.float32)

        pltpu.sync_copy(stage, buf_b.at[row_slice])
        plsc.subcore_barrier()

        ebase = s * edges_per_tile
        for r in range(ORDER):
            src_buf = buf_a if r % 2 == 0 else buf_b
            acc_buf = buf_b if r % 2 == 0 else buf_a

            @pl.loop(0, n_chunks)
            def _(ch):
                base = ebase + ch * CHUNK
                pltpu.sync_copy(src_hbm.at[pl.ds(base, CHUNK)], sidx)
                pltpu.sync_copy(dst_hbm.at[pl.ds(base, CHUNK)], didx)
                pltpu.sync_copy(w_hbm.at[pl.ds(base, CHUNK)], wbuf)
                pltpu.sync_copy(src_buf.at[sidx], rows)

                @pl.loop(0, CHUNK)
                def _(i):
                    wv = plsc.load_gather(wbuf, [jnp.full((L,), i, jnp.int32)])
                    for j in range(fh // L):
                        fs = pl.ds(j * L, L)
                        rows[i, fs] = rows[i, fs] * wv

                pltpu.sync_copy(rows, acc_buf.at[didx], add=True)

            plsc.subcore_barrier()

            pltpu.sync_copy(acc_buf.at[row_slice], stage)

            @pl.loop(0, rows_per_tile)
            def _(i):
                for j in range(fh // L):
                    fs = pl.ds(j * L, L)
                    prop[i, fs] = prop[i, fs] + stage[i, fs]
                    stage[i, fs] = jnp.zeros((L,), jnp.float32)

            if r < ORDER - 1:
                pltpu.sync_copy(stage, src_buf.at[row_slice])
                plsc.subcore_barrier()

        pltpu.sync_copy(prop, out_hbm.at[row_slice, pl.ds(fbase, fh)])

    return prop_kernel(x_pad, src, dst, w)


def _head_tc(s_arr, w1, b1, w2, b2):
    n_pad = s_arr.shape[0]
    c_out = w2.shape[1]

    def head_kernel(s_ref, w1_ref, b1_ref, w2_ref, b2_ref, o_ref):
        x = s_ref[...]
        nrm = jnp.sqrt(jnp.sum(x * x, axis=1, keepdims=True))
        x = x / (1e-12 + nrm)
        h = jnp.dot(x, w1_ref[...], preferred_element_type=jnp.float32)
        h = h + b1_ref[...]
        h = jnp.maximum(h, 0.0)
        hn = jnp.sqrt(jnp.sum(h * h, axis=1, keepdims=True))
        h = h / (1e-12 + hn)
        o = jnp.dot(h, w2_ref[...], preferred_element_type=jnp.float32)
        o_ref[...] = o + b2_ref[...]

    return pl.pallas_call(
        head_kernel,
        out_shape=jax.ShapeDtypeStruct((n_pad, c_out), jnp.float32),
    )(s_arr, w1, b1, w2, b2)


def kernel(X, edge_index, edge_weight, W1, b1, W2, b2):
    n, f = X.shape
    e = edge_weight.shape[0]
    n_pad = ((n + NS * L - 1) // (NS * L)) * (NS * L)
    step = NS * CHUNK
    e_pad = ((e + step - 1) // step) * step

    src = edge_index[0].astype(jnp.int32)
    dst = edge_index[1].astype(jnp.int32)
    w = edge_weight.astype(jnp.float32)
    if e_pad != e:
        pad = e_pad - e
        src = jnp.concatenate([src, jnp.zeros((pad,), jnp.int32)])
        dst = jnp.concatenate([dst, jnp.zeros((pad,), jnp.int32)])
        w = jnp.concatenate([w, jnp.zeros((pad,), jnp.float32)])
    x_pad = X if n_pad == n else jnp.pad(X, ((0, n_pad - n), (0, 0)))

    s_arr = _prop_sc(n_pad, f, e_pad, x_pad, src, dst, w)
    out = _head_tc(s_arr, W1, b1.reshape(1, -1), W2, b2.reshape(1, -1))
    return out[:n]


# speedup vs baseline: 2.3850x; 2.3850x over previous
import functools

import jax
import jax.numpy as jnp
from jax import lax
from jax.experimental import pallas as pl
from jax.experimental.pallas import tpu as pltpu
from jax.experimental.pallas import tpu_sc as plsc

NC = 2
NS = 16
L = 16
CHUNK = 128
ORDER = 8


def _prop_sc(n_pad, f, e_pad, x_pad, src, dst, w):
    fh = f // NC
    rows_per_tile = n_pad // NS
    edges_per_tile = e_pad // NS
    n_chunks = edges_per_tile // CHUNK
    n_rb = rows_per_tile // CHUNK
    mesh = plsc.VectorSubcoreMesh(
        core_axis_name="c", subcore_axis_name="s", num_cores=NC, num_subcores=NS
    )

    @functools.partial(
        pl.kernel,
        out_type=jax.ShapeDtypeStruct((n_pad, f), jnp.float32),
        mesh=mesh,
        compiler_params=pltpu.CompilerParams(
            use_tc_tiling_on_sc=False, needs_layout_passes=False
        ),
        scratch_types=[
            pltpu.VMEM_SHARED((n_pad, fh), jnp.float32),
            pltpu.HBM((NC, n_pad, fh), jnp.float32),
            pltpu.HBM((NC, n_pad, fh), jnp.float32),
            pltpu.VMEM((rows_per_tile, fh), jnp.float32),
            pltpu.VMEM((CHUNK, fh), jnp.float32),
            pltpu.VMEM((CHUNK,), jnp.int32),
            pltpu.VMEM((CHUNK,), jnp.int32),
            pltpu.VMEM((CHUNK,), jnp.float32),
        ],
    )
    def prop_kernel(x_hbm, src_hbm, dst_hbm, w_hbm, out_hbm,
                    acc, xca, xcb, prop, rows, sidx, didx, wbuf):
        c = lax.axis_index("c")
        s = lax.axis_index("s")
        fbase = c * fh
        r0 = s * rows_per_tile
        row_slice = pl.ds(r0, rows_per_tile)

        def zero_acc_slice():
            @pl.loop(0, CHUNK)
            def _(i):
                for j in range(fh // L):
                    rows[i, pl.ds(j * L, L)] = jnp.zeros((L,), jnp.float32)

            @pl.loop(0, n_rb)
            def _(b):
                pltpu.sync_copy(rows, acc.at[pl.ds(r0 + b * CHUNK, CHUNK)])

        pltpu.sync_copy(x_hbm.at[row_slice, pl.ds(fbase, fh)], prop)
        pltpu.sync_copy(prop, xca.at[c, row_slice])
        zero_acc_slice()
        plsc.subcore_barrier()

        ebase = s * edges_per_tile
        for r in range(ORDER):
            xsrc = xca if r % 2 == 0 else xcb
            xdst = xcb if r % 2 == 0 else xca

            @pl.loop(0, n_chunks)
            def _(ch):
                base = ebase + ch * CHUNK
                pltpu.sync_copy(src_hbm.at[pl.ds(base, CHUNK)], sidx)
                pltpu.sync_copy(dst_hbm.at[pl.ds(base, CHUNK)], didx)
                pltpu.sync_copy(w_hbm.at[pl.ds(base, CHUNK)], wbuf)
                pltpu.sync_copy(xsrc.at[c].at[sidx], rows)

                @pl.loop(0, CHUNK)
                def _(i):
                    wv = plsc.load_gather(wbuf, [jnp.full((L,), i, jnp.int32)])
                    for j in range(fh // L):
                        fs = pl.ds(j * L, L)
                        rows[i, fs] = rows[i, fs] * wv

                pltpu.sync_copy(rows, acc.at[didx], add=True)

            plsc.subcore_barrier()

            if r < ORDER - 1:
                pltpu.sync_copy(acc.at[row_slice], xdst.at[c, row_slice])

            @pl.loop(0, n_rb)
            def _(b):
                pltpu.sync_copy(acc.at[pl.ds(r0 + b * CHUNK, CHUNK)], rows)

                @pl.loop(0, CHUNK)
                def _(i):
                    for j in range(fh // L):
                        fs = pl.ds(j * L, L)
                        prop[b * CHUNK + i, fs] = prop[b * CHUNK + i, fs] + rows[i, fs]

            if r < ORDER - 1:
                zero_acc_slice()
                plsc.subcore_barrier()

        pltpu.sync_copy(prop, out_hbm.at[row_slice, pl.ds(fbase, fh)])

    return prop_kernel(x_pad, src, dst, w)


def _head_tc(s_arr, w1, b1, w2, b2):
    n_pad = s_arr.shape[0]
    c_out = w2.shape[1]

    def head_kernel(s_ref, w1_ref, b1_ref, w2_ref, b2_ref, o_ref):
        x = s_ref[...]
        nrm = jnp.sqrt(jnp.sum(x * x, axis=1, keepdims=True))
        x = x / (1e-12 + nrm)
        h = jnp.dot(x, w1_ref[...], preferred_element_type=jnp.float32)
        h = h + b1_ref[...]
        h = jnp.maximum(h, 0.0)
        hn = jnp.sqrt(jnp.sum(h * h, axis=1, keepdims=True))
        h = h / (1e-12 + hn)
        o = jnp.dot(h, w2_ref[...], preferred_element_type=jnp.float32)
        o_ref[...] = o + b2_ref[...]

    return pl.pallas_call(
        head_kernel,
        out_shape=jax.ShapeDtypeStruct((n_pad, c_out), jnp.float32),
    )(s_arr, w1, b1, w2, b2)


def kernel(X, edge_index, edge_weight, W1, b1, W2, b2):
    n, f = X.shape
    e = edge_weight.shape[0]
    n_pad = ((n + NS * L - 1) // (NS * L)) * (NS * L)
    step = NS * CHUNK
    e_pad = ((e + step - 1) // step) * step

    src = edge_index[0].astype(jnp.int32)
    dst = edge_index[1].astype(jnp.int32)
    w = edge_weight.astype(jnp.float32)
    if e_pad != e:
        pad = e_pad - e
        src = jnp.concatenate([src, jnp.zeros((pad,), jnp.int32)])
        dst = jnp.concatenate([dst, jnp.zeros((pad,), jnp.int32)])
        w = jnp.concatenate([w, jnp.zeros((pad,), jnp.float32)])
    x_pad = X if n_pad == n else jnp.pad(X, ((0, n_pad - n), (0, 0)))

    s_arr = _prop_sc(n_pad, f, e_pad, x_pad, src, dst, w)
    out = _head_tc(s_arr, W1, b1.reshape(1, -1), W2, b2.reshape(1, -1))
    return out[:n]


# speedup vs baseline: 3.2840x; 1.3769x over previous
import functools

import jax
import jax.numpy as jnp
from jax import lax
from jax.experimental import pallas as pl
from jax.experimental.pallas import tpu as pltpu
from jax.experimental.pallas import tpu_sc as plsc

NC = 2
NS = 16
L = 16
CHUNK = 128
GRP = 8
ORDER = 8


def _prop_sc(n_pad, f, e_pad, x_pad, src2, dst2, w2):
    fh = f // NC
    rows_per_tile = n_pad // NS
    chunks_per_tile = (e_pad // CHUNK) // NS
    n_groups = chunks_per_tile // GRP
    n_rb = rows_per_tile // CHUNK
    mesh = plsc.VectorSubcoreMesh(
        core_axis_name="c", subcore_axis_name="s", num_cores=NC, num_subcores=NS
    )

    @functools.partial(
        pl.kernel,
        out_type=jax.ShapeDtypeStruct((n_pad, f), jnp.float32),
        mesh=mesh,
        compiler_params=pltpu.CompilerParams(
            use_tc_tiling_on_sc=False, needs_layout_passes=False
        ),
        scratch_types=[
            pltpu.VMEM_SHARED((n_pad, fh), jnp.float32),
            pltpu.HBM((NC, n_pad, fh), jnp.float32),
            pltpu.HBM((NC, n_pad, fh), jnp.float32),
            pltpu.VMEM((rows_per_tile, fh), jnp.float32),
            pltpu.VMEM((CHUNK, fh), jnp.float32),
            pltpu.VMEM((CHUNK, fh), jnp.float32),
            pltpu.VMEM((GRP, CHUNK), jnp.int32),
            pltpu.VMEM((GRP, CHUNK), jnp.int32),
            pltpu.VMEM((GRP, CHUNK), jnp.float32),
            pltpu.SemaphoreType.DMA,
            pltpu.SemaphoreType.DMA,
            pltpu.SemaphoreType.DMA,
            pltpu.SemaphoreType.DMA,
            pltpu.SemaphoreType.DMA,
        ],
    )
    def prop_kernel(x_hbm, src_hbm, dst_hbm, w_hbm, out_hbm,
                    acc, xca, xcb, prop, rows_a, rows_b, sidx, didx, wbuf,
                    isem, gsem_a, gsem_b, ssem_a, ssem_b):
        c = lax.axis_index("c")
        s = lax.axis_index("s")
        fbase = c * fh
        r0 = s * rows_per_tile
        row_slice = pl.ds(r0, rows_per_tile)

        def zero_acc_slice():
            @pl.loop(0, CHUNK)
            def _(i):
                for j in range(fh // L):
                    rows_a[i, pl.ds(j * L, L)] = jnp.zeros((L,), jnp.float32)

            @pl.loop(0, n_rb)
            def _(b):
                pltpu.sync_copy(rows_a, acc.at[pl.ds(r0 + b * CHUNK, CHUNK)])

        pltpu.sync_copy(x_hbm.at[row_slice, pl.ds(fbase, fh)], prop)
        pltpu.sync_copy(prop, xca.at[c, row_slice])
        zero_acc_slice()
        plsc.subcore_barrier()

        cbase = s * chunks_per_tile
        gsems = (gsem_a, gsem_b)
        ssems = (ssem_a, ssem_b)
        bufs = (rows_a, rows_b)

        for r in range(ORDER):
            xsrc = xca if r % 2 == 0 else xcb
            xdst = xcb if r % 2 == 0 else xca

            @pl.loop(0, n_groups)
            def _(g):
                crow = cbase + g * GRP
                cps = pltpu.async_copy(src_hbm.at[pl.ds(crow, GRP)], sidx, isem)
                cpd = pltpu.async_copy(dst_hbm.at[pl.ds(crow, GRP)], didx, isem)
                cpw = pltpu.async_copy(w_hbm.at[pl.ds(crow, GRP)], wbuf, isem)
                cps.wait()
                cpd.wait()
                cpw.wait()

                gathers = [None] * GRP
                scats = [None] * GRP
                gathers[0] = pltpu.async_copy(
                    xsrc.at[c].at[sidx.at[0]], bufs[0], gsems[0])
                for k in range(GRP):
                    p = k % 2
                    cur = bufs[p]
                    gathers[k].wait()
                    if k + 1 < GRP:
                        if k >= 1:
                            scats[k - 1].wait()
                        gathers[k + 1] = pltpu.async_copy(
                            xsrc.at[c].at[sidx.at[k + 1]], bufs[1 - p],
                            gsems[1 - p])

                    @pl.loop(0, CHUNK, unroll=4)
                    def _(i):
                        wv = plsc.load_gather(
                            wbuf, [jnp.full((L,), k, jnp.int32),
                                   jnp.full((L,), i, jnp.int32)])
                        for j in range(fh // L):
                            fs = pl.ds(j * L, L)
                            cur[i, fs] = cur[i, fs] * wv

                    scats[k] = pltpu.async_copy(
                        cur, acc.at[didx.at[k]], ssems[p], add=True)

                scats[GRP - 2].wait()
                scats[GRP - 1].wait()

            plsc.subcore_barrier()

            if r < ORDER - 1:
                pltpu.sync_copy(acc.at[row_slice], xdst.at[c, row_slice])

            @pl.loop(0, n_rb)
            def _(b):
                pltpu.sync_copy(acc.at[pl.ds(r0 + b * CHUNK, CHUNK)], rows_a)

                @pl.loop(0, CHUNK)
                def _(i):
                    for j in range(fh // L):
                        fs = pl.ds(j * L, L)
                        prop[b * CHUNK + i, fs] = (
                            prop[b * CHUNK + i, fs] + rows_a[i, fs])

            if r < ORDER - 1:
                zero_acc_slice()
                plsc.subcore_barrier()

        pltpu.sync_copy(prop, out_hbm.at[row_slice, pl.ds(fbase, fh)])

    return prop_kernel(x_pad, src2, dst2, w2)


def _head_tc(s_arr, w1, b1, w2, b2):
    n_pad = s_arr.shape[0]
    c_out = w2.shape[1]

    def head_kernel(s_ref, w1_ref, b1_ref, w2_ref, b2_ref, o_ref):
        x = s_ref[...]
        nrm = jnp.sqrt(jnp.sum(x * x, axis=1, keepdims=True))
        x = x / (1e-12 + nrm)
        h = jnp.dot(x, w1_ref[...], preferred_element_type=jnp.float32)
        h = h + b1_ref[...]
        h = jnp.maximum(h, 0.0)
        hn = jnp.sqrt(jnp.sum(h * h, axis=1, keepdims=True))
        h = h / (1e-12 + hn)
        o = jnp.dot(h, w2_ref[...], preferred_element_type=jnp.float32)
        o_ref[...] = o + b2_ref[...]

    return pl.pallas_call(
        head_kernel,
        out_shape=jax.ShapeDtypeStruct((n_pad, c_out), jnp.float32),
    )(s_arr, w1, b1, w2, b2)


def kernel(X, edge_index, edge_weight, W1, b1, W2, b2):
    n, f = X.shape
    e = edge_weight.shape[0]
    n_pad = ((n + NS * L - 1) // (NS * L)) * (NS * L)
    step = NS * GRP * CHUNK
    e_pad = ((e + step - 1) // step) * step

    src = edge_index[0].astype(jnp.int32)
    dst = edge_index[1].astype(jnp.int32)
    w = edge_weight.astype(jnp.float32)
    if e_pad != e:
        pad = e_pad - e
        src = jnp.concatenate([src, jnp.zeros((pad,), jnp.int32)])
        dst = jnp.concatenate([dst, jnp.zeros((pad,), jnp.int32)])
        w = jnp.concatenate([w, jnp.zeros((pad,), jnp.float32)])
    src2 = src.reshape(e_pad // CHUNK, CHUNK)
    dst2 = dst.reshape(e_pad // CHUNK, CHUNK)
    w2 = w.reshape(e_pad // CHUNK, CHUNK)
    x_pad = X if n_pad == n else jnp.pad(X, ((0, n_pad - n), (0, 0)))

    s_arr = _prop_sc(n_pad, f, e_pad, x_pad, src2, dst2, w2)
    out = _head_tc(s_arr, W1, b1.reshape(1, -1), W2, b2.reshape(1, -1))
    return out[:n]


# speedup vs baseline: 6.0338x; 1.8373x over previous
import functools

import jax
import jax.numpy as jnp
from jax import lax
from jax.experimental import pallas as pl
from jax.experimental.pallas import tpu as pltpu
from jax.experimental.pallas import tpu_sc as plsc

NC = 2
NS = 16
L = 16
CHUNK = 128
GRP = 8
ORDER = 8


def _prop_sc(n_pad, f, e_pad, x_pad, src2, dst2, w2):
    fh = f // NC
    rows_per_tile = n_pad // NS
    chunks_per_tile = (e_pad // CHUNK) // NS
    n_groups = chunks_per_tile // GRP
    n_rb = rows_per_tile // CHUNK
    mesh = plsc.VectorSubcoreMesh(
        core_axis_name="c", subcore_axis_name="s", num_cores=NC, num_subcores=NS
    )

    @functools.partial(
        pl.kernel,
        out_type=jax.ShapeDtypeStruct((n_pad, f), jnp.float32),
        mesh=mesh,
        compiler_params=pltpu.CompilerParams(
            use_tc_tiling_on_sc=False, needs_layout_passes=False
        ),
        scratch_types=[
            pltpu.VMEM_SHARED((n_pad, fh), jnp.float32),
            pltpu.VMEM_SHARED((n_pad, fh), jnp.float32),
            pltpu.VMEM((CHUNK, fh), jnp.float32),
            pltpu.VMEM((CHUNK, fh), jnp.float32),
            pltpu.VMEM((GRP, CHUNK), jnp.int32),
            pltpu.VMEM((GRP, CHUNK), jnp.int32),
            pltpu.VMEM((GRP, CHUNK), jnp.float32),
            pltpu.SemaphoreType.DMA,
            pltpu.SemaphoreType.DMA,
            pltpu.SemaphoreType.DMA,
            pltpu.SemaphoreType.DMA,
            pltpu.SemaphoreType.DMA,
        ],
    )
    def prop_kernel(x_hbm, src_hbm, dst_hbm, w_hbm, out_hbm,
                    buf_a, buf_b, rows_a, rows_b, sidx, didx, wbuf,
                    isem, gsem_a, gsem_b, ssem_a, ssem_b):
        c = lax.axis_index("c")
        s = lax.axis_index("s")
        fbase = c * fh
        r0 = s * rows_per_tile
        row_slice = pl.ds(r0, rows_per_tile)
        fcol = pl.ds(fbase, fh)

        def zero_rows_a():
            @pl.loop(0, CHUNK)
            def _(i):
                for j in range(fh // L):
                    rows_a[i, pl.ds(j * L, L)] = jnp.zeros((L,), jnp.float32)

        pltpu.sync_copy(x_hbm.at[row_slice, fcol], buf_a.at[row_slice])

        @pl.loop(0, n_rb)
        def _(b):
            blk = pl.ds(r0 + b * CHUNK, CHUNK)
            pltpu.sync_copy(x_hbm.at[blk, fcol], rows_b)
            pltpu.sync_copy(rows_b, out_hbm.at[blk, fcol])

        zero_rows_a()

        @pl.loop(0, n_rb)
        def _(b):
            pltpu.sync_copy(rows_a, buf_b.at[pl.ds(r0 + b * CHUNK, CHUNK)])

        plsc.subcore_barrier()

        cbase = s * chunks_per_tile
        gsems = (gsem_a, gsem_b)
        ssems = (ssem_a, ssem_b)
        bufs = (rows_a, rows_b)

        for r in range(ORDER):
            xsrc = buf_a if r % 2 == 0 else buf_b
            acc = buf_b if r % 2 == 0 else buf_a

            @pl.loop(0, n_groups)
            def _(g):
                crow = cbase + g * GRP
                cps = pltpu.async_copy(src_hbm.at[pl.ds(crow, GRP)], sidx, isem)
                cpd = pltpu.async_copy(dst_hbm.at[pl.ds(crow, GRP)], didx, isem)
                cpw = pltpu.async_copy(w_hbm.at[pl.ds(crow, GRP)], wbuf, isem)
                cps.wait()
                cpd.wait()
                cpw.wait()

                gathers = [None] * GRP
                scats = [None] * GRP
                gathers[0] = pltpu.async_copy(
                    xsrc.at[sidx.at[0]], bufs[0], gsems[0])
                for k in range(GRP):
                    p = k % 2
                    cur = bufs[p]
                    gathers[k].wait()
                    if k + 1 < GRP:
                        if k >= 1:
                            scats[k - 1].wait()
                        gathers[k + 1] = pltpu.async_copy(
                            xsrc.at[sidx.at[k + 1]], bufs[1 - p],
                            gsems[1 - p])

                    @pl.loop(0, CHUNK, unroll=4)
                    def _(i):
                        wv = plsc.load_gather(
                            wbuf, [jnp.full((L,), k, jnp.int32),
                                   jnp.full((L,), i, jnp.int32)])
                        for j in range(fh // L):
                            fs = pl.ds(j * L, L)
                            cur[i, fs] = cur[i, fs] * wv

                    scats[k] = pltpu.async_copy(
                        cur, acc.at[didx.at[k]], ssems[p], add=True)

                scats[GRP - 2].wait()
                scats[GRP - 1].wait()

            plsc.subcore_barrier()

            @pl.loop(0, n_rb)
            def _(b):
                blk = pl.ds(r0 + b * CHUNK, CHUNK)
                pltpu.sync_copy(acc.at[blk], rows_a)
                pltpu.sync_copy(out_hbm.at[blk, fcol], rows_b)

                @pl.loop(0, CHUNK)
                def _(i):
                    for j in range(fh // L):
                        fs = pl.ds(j * L, L)
                        rows_b[i, fs] = rows_b[i, fs] + rows_a[i, fs]

                pltpu.sync_copy(rows_b, out_hbm.at[blk, fcol])

            if r < ORDER - 1:
                zero_rows_a()

                @pl.loop(0, n_rb)
                def _(b):
                    pltpu.sync_copy(rows_a, xsrc.at[pl.ds(r0 + b * CHUNK, CHUNK)])

                plsc.subcore_barrier()

    return prop_kernel(x_pad, src2, dst2, w2)


def _head_tc(s_arr, w1, b1, w2, b2):
    n_pad = s_arr.shape[0]
    c_out = w2.shape[1]

    def head_kernel(s_ref, w1_ref, b1_ref, w2_ref, b2_ref, o_ref):
        x = s_ref[...]
        nrm = jnp.sqrt(jnp.sum(x * x, axis=1, keepdims=True))
        x = x / (1e-12 + nrm)
        h = jnp.dot(x, w1_ref[...], preferred_element_type=jnp.float32)
        h = h + b1_ref[...]
        h = jnp.maximum(h, 0.0)
        hn = jnp.sqrt(jnp.sum(h * h, axis=1, keepdims=True))
        h = h / (1e-12 + hn)
        o = jnp.dot(h, w2_ref[...], preferred_element_type=jnp.float32)
        o_ref[...] = o + b2_ref[...]

    return pl.pallas_call(
        head_kernel,
        out_shape=jax.ShapeDtypeStruct((n_pad, c_out), jnp.float32),
    )(s_arr, w1, b1, w2, b2)


def kernel(X, edge_index, edge_weight, W1, b1, W2, b2):
    n, f = X.shape
    e = edge_weight.shape[0]
    n_pad = ((n + NS * L - 1) // (NS * L)) * (NS * L)
    step = NS * GRP * CHUNK
    e_pad = ((e + step - 1) // step) * step

    src = edge_index[0].astype(jnp.int32)
    dst = edge_index[1].astype(jnp.int32)
    w = edge_weight.astype(jnp.float32)
    if e_pad != e:
        pad = e_pad - e
        src = jnp.concatenate([src, jnp.zeros((pad,), jnp.int32)])
        dst = jnp.concatenate([dst, jnp.zeros((pad,), jnp.int32)])
        w = jnp.concatenate([w, jnp.zeros((pad,), jnp.float32)])
    src2 = src.reshape(e_pad // CHUNK, CHUNK)
    dst2 = dst.reshape(e_pad // CHUNK, CHUNK)
    w2 = w.reshape(e_pad // CHUNK, CHUNK)
    x_pad = X if n_pad == n else jnp.pad(X, ((0, n_pad - n), (0, 0)))

    s_arr = _prop_sc(n_pad, f, e_pad, x_pad, src2, dst2, w2)
    out = _head_tc(s_arr, W1, b1.reshape(1, -1), W2, b2.reshape(1, -1))
    return out[:n]


# speedup vs baseline: 6.9113x; 1.1454x over previous
import functools

import jax
import jax.numpy as jnp
from jax import lax
from jax.experimental import pallas as pl
from jax.experimental.pallas import tpu as pltpu
from jax.experimental.pallas import tpu_sc as plsc

NC = 2
NS = 16
L = 16
CHUNK = 128
GRP = 8
ORDER = 8


def _prop_sc(n_pad, f, e_pad, x_pad, src2, dst2, w2):
    fh = f // NC
    rows_per_tile = n_pad // NS
    chunks_per_tile = (e_pad // CHUNK) // NS
    n_groups = chunks_per_tile // GRP
    n_rb = rows_per_tile // CHUNK
    mesh = plsc.VectorSubcoreMesh(
        core_axis_name="c", subcore_axis_name="s", num_cores=NC, num_subcores=NS
    )

    @functools.partial(
        pl.kernel,
        out_type=jax.ShapeDtypeStruct((n_pad, f), jnp.float32),
        mesh=mesh,
        compiler_params=pltpu.CompilerParams(
            use_tc_tiling_on_sc=False, needs_layout_passes=False
        ),
        scratch_types=[
            pltpu.VMEM_SHARED((n_pad, fh), jnp.float32),
            pltpu.VMEM_SHARED((n_pad, fh), jnp.float32),
            pltpu.VMEM((CHUNK, fh), jnp.float32),
            pltpu.VMEM((CHUNK, fh), jnp.float32),
            pltpu.VMEM((CHUNK, fh), jnp.float32),
            pltpu.VMEM((CHUNK, fh), jnp.float32),
            pltpu.VMEM((GRP, CHUNK), jnp.int32),
            pltpu.VMEM((GRP, CHUNK), jnp.int32),
            pltpu.VMEM((GRP, CHUNK), jnp.float32),
            pltpu.VMEM((GRP, CHUNK), jnp.int32),
            pltpu.VMEM((GRP, CHUNK), jnp.int32),
            pltpu.VMEM((GRP, CHUNK), jnp.float32),
            pltpu.SemaphoreType.DMA,
            pltpu.SemaphoreType.DMA,
            pltpu.SemaphoreType.DMA,
            pltpu.SemaphoreType.DMA,
            pltpu.SemaphoreType.DMA,
            pltpu.SemaphoreType.DMA,
            pltpu.SemaphoreType.DMA,
            pltpu.SemaphoreType.DMA,
            pltpu.SemaphoreType.DMA,
            pltpu.SemaphoreType.DMA,
        ],
    )
    def prop_kernel(x_hbm, src_hbm, dst_hbm, w_hbm, out_hbm,
                    buf_a, buf_b, rows_0, rows_1, rows_2, rows_3,
                    sidx0, didx0, wbuf0, sidx1, didx1, wbuf1,
                    isem0, isem1, gsem_0, gsem_1, gsem_2, gsem_3,
                    ssem_0, ssem_1, ssem_2, ssem_3):
        c = lax.axis_index("c")
        s = lax.axis_index("s")
        fbase = c * fh
        r0 = s * rows_per_tile
        row_slice = pl.ds(r0, rows_per_tile)
        fcol = pl.ds(fbase, fh)

        def zero_rows_0():
            @pl.loop(0, CHUNK)
            def _(i):
                for j in range(fh // L):
                    rows_0[i, pl.ds(j * L, L)] = jnp.zeros((L,), jnp.float32)

        pltpu.sync_copy(x_hbm.at[row_slice, fcol], buf_a.at[row_slice])

        @pl.loop(0, n_rb)
        def _(b):
            blk = pl.ds(r0 + b * CHUNK, CHUNK)
            pltpu.sync_copy(x_hbm.at[blk, fcol], rows_1)
            pltpu.sync_copy(rows_1, out_hbm.at[blk, fcol])

        zero_rows_0()

        @pl.loop(0, n_rb)
        def _(b):
            pltpu.sync_copy(rows_0, buf_b.at[pl.ds(r0 + b * CHUNK, CHUNK)])

        plsc.subcore_barrier()

        cbase = s * chunks_per_tile
        rows = (rows_0, rows_1, rows_2, rows_3)
        gsems = (gsem_0, gsem_1, gsem_2, gsem_3)
        ssems = (ssem_0, ssem_1, ssem_2, ssem_3)
        idx0 = (sidx0, didx0, wbuf0, isem0)
        idx1 = (sidx1, didx1, wbuf1, isem1)
        last_row = cbase + (n_groups - 1) * GRP

        def load_idx(g_row, bufs):
            sb, db, wb, sem = bufs
            pltpu.async_copy(src_hbm.at[pl.ds(g_row, GRP)], sb, sem)
            pltpu.async_copy(dst_hbm.at[pl.ds(g_row, GRP)], db, sem)
            pltpu.async_copy(w_hbm.at[pl.ds(g_row, GRP)], wb, sem)

        def wait_idx(bufs):
            sb, db, wb, sem = bufs
            pltpu.make_async_copy(src_hbm.at[pl.ds(0, GRP)], sb, sem).wait()
            pltpu.make_async_copy(dst_hbm.at[pl.ds(0, GRP)], db, sem).wait()
            pltpu.make_async_copy(w_hbm.at[pl.ds(0, GRP)], wb, sem).wait()

        def process_group(xsrc, acc, bufs):
            sb, db, wb, _ = bufs
            gathers = [None] * GRP
            scats = [None] * GRP
            for k in range(2):
                gathers[k] = pltpu.async_copy(
                    xsrc.at[sb.at[k]], rows[k], gsems[k])
            for k in range(GRP):
                q = k % 4
                cur = rows[q]
                gathers[k].wait()
                if k + 2 < GRP:
                    if k >= 2:
                        scats[k - 2].wait()
                    gathers[k + 2] = pltpu.async_copy(
                        xsrc.at[sb.at[k + 2]], rows[(k + 2) % 4],
                        gsems[(k + 2) % 4])

                @pl.loop(0, CHUNK, unroll=4)
                def _(i):
                    wv = plsc.load_gather(
                        wb, [jnp.full((L,), k, jnp.int32),
                             jnp.full((L,), i, jnp.int32)])
                    for j in range(fh // L):
                        fs = pl.ds(j * L, L)
                        cur[i, fs] = cur[i, fs] * wv

                scats[k] = pltpu.async_copy(
                    cur, acc.at[db.at[k]], ssems[q], add=True)

            for k in range(GRP - 4, GRP):
                scats[k].wait()

        def do_round(xsrc, acc):
            load_idx(cbase, idx0)
            load_idx(cbase + GRP, idx1)

            @pl.loop(0, n_groups // 2)
            def _(t):
                g0row = cbase + (2 * t) * GRP
                wait_idx(idx0)
                process_group(xsrc, acc, idx0)
                load_idx(jnp.minimum(g0row + 2 * GRP, last_row), idx0)
                wait_idx(idx1)
                process_group(xsrc, acc, idx1)
                load_idx(jnp.minimum(g0row + 3 * GRP, last_row), idx1)

            wait_idx(idx0)
            wait_idx(idx1)
            plsc.subcore_barrier()

            @pl.loop(0, n_rb)
            def _(b):
                blk = pl.ds(r0 + b * CHUNK, CHUNK)
                pltpu.sync_copy(acc.at[blk], rows_0)
                pltpu.sync_copy(out_hbm.at[blk, fcol], rows_1)

                @pl.loop(0, CHUNK)
                def _(i):
                    for j in range(fh // L):
                        fs = pl.ds(j * L, L)
                        rows_1[i, fs] = rows_1[i, fs] + rows_0[i, fs]

                pltpu.sync_copy(rows_1, out_hbm.at[blk, fcol])

            zero_rows_0()

            @pl.loop(0, n_rb)
            def _(b):
                pltpu.sync_copy(rows_0, xsrc.at[pl.ds(r0 + b * CHUNK, CHUNK)])

            plsc.subcore_barrier()

        @pl.loop(0, ORDER // 2)
        def _(rr):
            do_round(buf_a, buf_b)
            do_round(buf_b, buf_a)

    return prop_kernel(x_pad, src2, dst2, w2)


def _head_tc(s_arr, w1, b1, w2, b2):
    n_pad = s_arr.shape[0]
    c_out = w2.shape[1]

    def head_kernel(s_ref, w1_ref, b1_ref, w2_ref, b2_ref, o_ref):
        x = s_ref[...]
        nrm = jnp.sqrt(jnp.sum(x * x, axis=1, keepdims=True))
        x = x / (1e-12 + nrm)
        h = jnp.dot(x, w1_ref[...], preferred_element_type=jnp.float32)
        h = h + b1_ref[...]
        h = jnp.maximum(h, 0.0)
        hn = jnp.sqrt(jnp.sum(h * h, axis=1, keepdims=True))
        h = h / (1e-12 + hn)
        o = jnp.dot(h, w2_ref[...], preferred_element_type=jnp.float32)
        o_ref[...] = o + b2_ref[...]

    return pl.pallas_call(
        head_kernel,
        out_shape=jax.ShapeDtypeStruct((n_pad, c_out), jnp.float32),
    )(s_arr, w1, b1, w2, b2)


def kernel(X, edge_index, edge_weight, W1, b1, W2, b2):
    n, f = X.shape
    e = edge_weight.shape[0]
    n_pad = ((n + NS * L - 1) // (NS * L)) * (NS * L)
    step = NS * GRP * CHUNK
    e_pad = ((e + step - 1) // step) * step

    src = edge_index[0].astype(jnp.int32)
    dst = edge_index[1].astype(jnp.int32)
    w = edge_weight.astype(jnp.float32)
    if e_pad != e:
        pad = e_pad - e
        src = jnp.concatenate([src, jnp.zeros((pad,), jnp.int32)])
        dst = jnp.concatenate([dst, jnp.zeros((pad,), jnp.int32)])
        w = jnp.concatenate([w, jnp.zeros((pad,), jnp.float32)])
    src2 = src.reshape(e_pad // CHUNK, CHUNK)
    dst2 = dst.reshape(e_pad // CHUNK, CHUNK)
    w2 = w.reshape(e_pad // CHUNK, CHUNK)
    x_pad = X if n_pad == n else jnp.pad(X, ((0, n_pad - n), (0, 0)))

    s_arr = _prop_sc(n_pad, f, e_pad, x_pad, src2, dst2, w2)
    out = _head_tc(s_arr, W1, b1.reshape(1, -1), W2, b2.reshape(1, -1))
    return out[:n]


# speedup vs baseline: 7.7477x; 1.1210x over previous
import functools

import jax
import jax.numpy as jnp
from jax import lax
from jax.experimental import pallas as pl
from jax.experimental.pallas import tpu as pltpu
from jax.experimental.pallas import tpu_sc as plsc

NC = 2
NS = 16
L = 16
CHUNK = 128
GRP = 8
ORDER = 8


def _prop_sc(n_pad, f, e_pad, x_pad, src2, dst2, w2):
    fh = f // NC
    rows_per_tile = n_pad // NS
    chunks_per_tile = (e_pad // CHUNK) // NS
    n_groups = chunks_per_tile // GRP
    n_rb = rows_per_tile // CHUNK
    mesh = plsc.VectorSubcoreMesh(
        core_axis_name="c", subcore_axis_name="s", num_cores=NC, num_subcores=NS
    )

    @functools.partial(
        pl.kernel,
        out_type=jax.ShapeDtypeStruct((ORDER, n_pad, f), jnp.float32),
        mesh=mesh,
        compiler_params=pltpu.CompilerParams(
            use_tc_tiling_on_sc=False, needs_layout_passes=False
        ),
        scratch_types=[
            pltpu.VMEM_SHARED((2 * n_pad, fh), jnp.float32),
            pltpu.VMEM((CHUNK, fh), jnp.float32),
            pltpu.VMEM((CHUNK, fh), jnp.float32),
            pltpu.VMEM((CHUNK, fh), jnp.float32),
            pltpu.VMEM((CHUNK, fh), jnp.float32),
            pltpu.VMEM((GRP, CHUNK), jnp.int32),
            pltpu.VMEM((GRP, CHUNK), jnp.int32),
            pltpu.VMEM((GRP, CHUNK), jnp.float32),
            pltpu.VMEM((GRP, CHUNK), jnp.int32),
            pltpu.VMEM((GRP, CHUNK), jnp.int32),
            pltpu.VMEM((GRP, CHUNK), jnp.float32),
            pltpu.SemaphoreType.DMA,
            pltpu.SemaphoreType.DMA,
            pltpu.SemaphoreType.DMA,
            pltpu.SemaphoreType.DMA,
            pltpu.SemaphoreType.DMA,
            pltpu.SemaphoreType.DMA,
            pltpu.SemaphoreType.DMA,
            pltpu.SemaphoreType.DMA,
            pltpu.SemaphoreType.DMA,
            pltpu.SemaphoreType.DMA,
        ],
    )
    def prop_kernel(x_hbm, src_hbm, dst_hbm, w_hbm, xs_hbm,
                    xbuf, rows_0, rows_1, rows_2, rows_3,
                    sidx0, didx0, wbuf0, sidx1, didx1, wbuf1,
                    isem0, isem1, gsem_0, gsem_1, gsem_2, gsem_3,
                    ssem_0, ssem_1, ssem_2, ssem_3):
        c = lax.axis_index("c")
        s = lax.axis_index("s")
        fbase = c * fh
        r0 = s * rows_per_tile
        row_slice = pl.ds(r0, rows_per_tile)
        fcol = pl.ds(fbase, fh)

        def zero_rows_0():
            @pl.loop(0, CHUNK)
            def _(i):
                for j in range(fh // L):
                    rows_0[i, pl.ds(j * L, L)] = jnp.zeros((L,), jnp.float32)

        pltpu.sync_copy(x_hbm.at[row_slice, fcol], xbuf.at[row_slice])
        zero_rows_0()

        @pl.loop(0, n_rb)
        def _(b):
            pltpu.sync_copy(rows_0, xbuf.at[pl.ds(n_pad + r0 + b * CHUNK, CHUNK)])

        plsc.subcore_barrier()

        cbase = s * chunks_per_tile
        rows = (rows_0, rows_1, rows_2, rows_3)
        gsems = (gsem_0, gsem_1, gsem_2, gsem_3)
        ssems = (ssem_0, ssem_1, ssem_2, ssem_3)
        idx0 = (sidx0, didx0, wbuf0, isem0)
        idx1 = (sidx1, didx1, wbuf1, isem1)
        last_row = cbase + (n_groups - 1) * GRP

        def load_idx(g_row, bufs):
            sb, db, wb, sem = bufs
            pltpu.async_copy(src_hbm.at[pl.ds(g_row, GRP)], sb, sem)
            pltpu.async_copy(dst_hbm.at[pl.ds(g_row, GRP)], db, sem)
            pltpu.async_copy(w_hbm.at[pl.ds(g_row, GRP)], wb, sem)

        def wait_idx(bufs, src_off, dst_off):
            sb, db, wb, sem = bufs
            pltpu.make_async_copy(src_hbm.at[pl.ds(0, GRP)], sb, sem).wait()
            pltpu.make_async_copy(dst_hbm.at[pl.ds(0, GRP)], db, sem).wait()
            pltpu.make_async_copy(w_hbm.at[pl.ds(0, GRP)], wb, sem).wait()
            so = jnp.full((L,), src_off, jnp.int32)
            do = jnp.full((L,), dst_off, jnp.int32)

            @pl.loop(0, GRP)
            def _(kk):
                for jj in range(CHUNK // L):
                    sl = pl.ds(jj * L, L)
                    sb[kk, sl] = sb[kk, sl] + so
                    db[kk, sl] = db[kk, sl] + do

        def process_group(bufs):
            sb, db, wb, _ = bufs
            gathers = [None] * GRP
            scats = [None] * GRP
            for k in range(2):
                gathers[k] = pltpu.async_copy(
                    xbuf.at[sb.at[k]], rows[k], gsems[k])
            for k in range(GRP):
                q = k % 4
                cur = rows[q]
                gathers[k].wait()
                if k + 2 < GRP:
                    if k >= 2:
                        scats[k - 2].wait()
                    gathers[k + 2] = pltpu.async_copy(
                        xbuf.at[sb.at[k + 2]], rows[(k + 2) % 4],
                        gsems[(k + 2) % 4])

                @pl.loop(0, CHUNK, step=L)
                def _(i0):
                    wv16 = wb[k, pl.ds(i0, L)]
                    for ii in range(L):
                        wv = jnp.full((L,), wv16[ii], jnp.float32)
                        for j in range(fh // L):
                            fs = pl.ds(j * L, L)
                            cur[i0 + ii, fs] = cur[i0 + ii, fs] * wv

                scats[k] = pltpu.async_copy(
                    cur, xbuf.at[db.at[k]], ssems[q], add=True)

            for k in range(GRP - 4, GRP):
                scats[k].wait()

        @pl.loop(0, ORDER)
        def _(r):
            ps = lax.rem(r, 2)
            src_off = ps * n_pad
            dst_off = (1 - ps) * n_pad

            load_idx(cbase, idx0)
            load_idx(cbase + GRP, idx1)

            @pl.loop(0, n_groups // 2)
            def _(t):
                g0row = cbase + (2 * t) * GRP
                wait_idx(idx0, src_off, dst_off)
                process_group(idx0)
                load_idx(jnp.minimum(g0row + 2 * GRP, last_row), idx0)
                wait_idx(idx1, src_off, dst_off)
                process_group(idx1)
                load_idx(jnp.minimum(g0row + 3 * GRP, last_row), idx1)

            wait_idx(idx0, 0, 0)
            wait_idx(idx1, 0, 0)
            plsc.subcore_barrier()

            pltpu.sync_copy(xbuf.at[pl.ds(dst_off + r0, rows_per_tile)],
                            xs_hbm.at[r, row_slice, fcol])
            zero_rows_0()

            @pl.loop(0, n_rb)
            def _(b):
                pltpu.sync_copy(
                    rows_0, xbuf.at[pl.ds(src_off + r0 + b * CHUNK, CHUNK)])

            plsc.subcore_barrier()

    return prop_kernel(x_pad, src2, dst2, w2)


def _head_tc(x_pad, xs, w1, b1, w2, b2):
    n_pad, f = x_pad.shape
    order = xs.shape[0]
    hid = w1.shape[1]
    c_out = w2.shape[1]
    br = 1280

    def head_kernel(x_ref, xs_ref, w1_ref, b1_ref, w2_ref, b2_ref, o_ref):
        x = x_ref[...] + jnp.sum(xs_ref[...], axis=0)
        nrm = jnp.sqrt(jnp.sum(x * x, axis=1, keepdims=True))
        x = x / (1e-12 + nrm)
        h = jnp.dot(x, w1_ref[...], preferred_element_type=jnp.float32)
        h = h + b1_ref[...]
        h = jnp.maximum(h, 0.0)
        hn = jnp.sqrt(jnp.sum(h * h, axis=1, keepdims=True))
        h = h / (1e-12 + hn)
        o = jnp.dot(h, w2_ref[...], preferred_element_type=jnp.float32)
        o_ref[...] = o + b2_ref[...]

    return pl.pallas_call(
        head_kernel,
        grid=(n_pad // br,),
        in_specs=[
            pl.BlockSpec((br, f), lambda i: (i, 0)),
            pl.BlockSpec((order, br, f), lambda i: (0, i, 0)),
            pl.BlockSpec((f, hid), lambda i: (0, 0)),
            pl.BlockSpec((1, hid), lambda i: (0, 0)),
            pl.BlockSpec((hid, c_out), lambda i: (0, 0)),
            pl.BlockSpec((1, c_out), lambda i: (0, 0)),
        ],
        out_specs=pl.BlockSpec((br, c_out), lambda i: (i, 0)),
        out_shape=jax.ShapeDtypeStruct((n_pad, c_out), jnp.float32),
    )(x_pad, xs, w1, b1, w2, b2)


def kernel(X, edge_index, edge_weight, W1, b1, W2, b2):
    n, f = X.shape
    e = edge_weight.shape[0]
    n_pad = ((n + NS * L - 1) // (NS * L)) * (NS * L)
    step = NS * GRP * CHUNK
    e_pad = ((e + step - 1) // step) * step

    src = edge_index[0].astype(jnp.int32)
    dst = edge_index[1].astype(jnp.int32)
    w = edge_weight.astype(jnp.float32)
    if e_pad != e:
        pad = e_pad - e
        src = jnp.concatenate([src, jnp.zeros((pad,), jnp.int32)])
        dst = jnp.concatenate([dst, jnp.zeros((pad,), jnp.int32)])
        w = jnp.concatenate([w, jnp.zeros((pad,), jnp.float32)])
    src2 = src.reshape(e_pad // CHUNK, CHUNK)
    dst2 = dst.reshape(e_pad // CHUNK, CHUNK)
    w2 = w.reshape(e_pad // CHUNK, CHUNK)
    x_pad = X if n_pad == n else jnp.pad(X, ((0, n_pad - n), (0, 0)))

    xs = _prop_sc(n_pad, f, e_pad, x_pad, src2, dst2, w2)
    out = _head_tc(x_pad, xs, W1, b1.reshape(1, -1), W2, b2.reshape(1, -1))
    return out[:n]
